# baseline (speedup 1.0000x reference)
import functools

import jax
import jax.numpy as jnp
from jax import lax
from jax.experimental import pallas as pl
from jax.experimental.pallas import tpu as pltpu
from jax.experimental.pallas import tpu_sc as plsc

_N = 10000
_D = 128
_E = 320000
_G = 64
_LANES = 16
_CHUNK = 64
_NCH = _E // _CHUNK
_NSLOT = 3
_NW = 32
_STRIPE = 624
_TAIL = _N - 16 * _STRIPE

_DP = _D // 2


def _pack_rows(h):
    bits = lax.bitcast_convert_type(h, jnp.int32)
    r = (bits + 32768) >> 16
    return (r[:, 0:_DP] & 65535) | (r[:, _DP:_D] << 16)


def _edge_mlp_body(ea_ref, w1_ref, b1_ref, w2_ref, b2_ref, w3_ref, b3_ref,
                   o1_ref, o2_ref, o3_ref):
    ea = ea_ref[...]
    o1_ref[...] = _pack_rows(
        jnp.dot(ea, w1_ref[...], preferred_element_type=jnp.float32)
        + b1_ref[...])
    o2_ref[...] = _pack_rows(
        jnp.dot(ea, w2_ref[...], preferred_element_type=jnp.float32)
        + b2_ref[...])
    o3_ref[...] = _pack_rows(
        jnp.dot(ea, w3_ref[...], preferred_element_type=jnp.float32)
        + b3_ref[...])


def _edge_mlp(edge_attr, w1, b1, w2, b2, w3, b3):
    blk = 2000
    grid = _E // blk
    ed = edge_attr.shape[1]
    out_spec = pl.BlockSpec((blk, _DP), lambda i: (i, 0))
    w_spec = pl.BlockSpec((ed, _D), lambda i: (0, 0))
    b_spec = pl.BlockSpec((1, _D), lambda i: (0, 0))
    return pl.pallas_call(
        _edge_mlp_body,
        grid=(grid,),
        in_specs=[pl.BlockSpec((blk, ed), lambda i: (i, 0)),
                  w_spec, b_spec, w_spec, b_spec, w_spec, b_spec],
        out_specs=[out_spec, out_spec, out_spec],
        out_shape=[jax.ShapeDtypeStruct((_E, _DP), jnp.int32)] * 3,
    )(edge_attr, w1, b1.reshape(1, _D), w2, b2.reshape(1, _D),
      w3, b3.reshape(1, _D))


def _xpack_body(x_ref, o_ref):
    o_ref[...] = _pack_rows(x_ref[...])


def _xpack(x):
    blk = 2000
    return pl.pallas_call(
        _xpack_body,
        grid=(_N // blk,),
        in_specs=[pl.BlockSpec((blk, _D), lambda i: (i, 0))],
        out_specs=pl.BlockSpec((blk, _DP), lambda i: (i, 0)),
        out_shape=jax.ShapeDtypeStruct((_N, _DP), jnp.int32),
    )(x)


def _sc_edge_body(x_hbm, e_hbm, src_hbm, dst_hbm, out_hbm, *sc):
    src_v = sc[0:3]
    dst_v = sc[3:6]
    xbuf = sc[6:9]
    ebuf = sc[9:12]
    mbuf = sc[12:15]
    acc = sc[15]
    sem_is = sc[16:19]
    sem_id = sc[19:22]
    sem_g = sc[22:25]
    sem_e = sc[25:28]
    sem_sc = sc[28:31]

    cid = lax.axis_index("c")
    sid = lax.axis_index("s")
    wid = sid * 2 + cid

    def zero_row(r, _):
        for j in range(_D // _LANES):
            mbuf[0][r, pl.ds(j * _LANES, _LANES)] = jnp.zeros(
                (_LANES,), jnp.float32)
        return 0

    lax.fori_loop(0, _CHUNK, zero_row, 0)
    for k in range(9):
        pltpu.sync_copy(mbuf[0], acc.at[pl.ds(sid * _STRIPE + k * _CHUNK,
                                              _CHUNK)])
    pltpu.sync_copy(mbuf[0].at[pl.ds(0, _STRIPE - 9 * _CHUNK)],
                    acc.at[pl.ds(sid * _STRIPE + 9 * _CHUNK,
                                 _STRIPE - 9 * _CHUNK)])

    @pl.when(sid == 15)
    def _zero_tail():
        pltpu.sync_copy(mbuf[0].at[pl.ds(0, _TAIL)],
                        acc.at[pl.ds(16 * _STRIPE, _TAIL)])

    plsc.subcore_barrier()

    nch = 156 + jnp.where(wid < _NCH - 156 * _NW, 1, 0)

    def issue_loads(k, slot):
        base = (k * _NW + wid) * _CHUNK
        pltpu.async_copy(src_hbm.at[pl.ds(base, _CHUNK)], src_v[slot],
                         sem_is[slot])
        pltpu.async_copy(dst_hbm.at[pl.ds(base, _CHUNK)], dst_v[slot],
                         sem_id[slot])
        pltpu.async_copy(e_hbm.at[pl.ds(base, _CHUNK)], ebuf[slot],
                         sem_e[slot])

    def issue_gather(slot):
        pltpu.make_async_copy(src_hbm.at[pl.ds(0, _CHUNK)], src_v[slot],
                              sem_is[slot]).wait()
        pltpu.async_copy(x_hbm.at[src_v[slot]], xbuf[slot], sem_g[slot])

    issue_loads(jnp.int32(0), 0)
    issue_loads(jnp.int32(1), 1)
    issue_gather(0)

    def tri_body(i3, _):
        for u in range(_NSLOT):
            k = i3 * _NSLOT + u
            s1 = (u + 1) % _NSLOT
            sj = (u + 2) % _NSLOT
            j = k + 2

            @pl.when((k >= 1) & (j < nch))
            def _refill_wait():
                pltpu.make_async_copy(mbuf[sj], acc.at[pl.ds(0, _CHUNK)],
                                      sem_sc[sj]).wait()
                issue_loads(j, sj)

            @pl.when((k == 0) & (j < nch))
            def _refill_fresh():
                issue_loads(j, sj)

            @pl.when(k + 1 < nch)
            def _gather_next():
                issue_gather(s1)

            @pl.when(k < nch)
            def _process():
                pltpu.make_async_copy(x_hbm.at[pl.ds(0, _CHUNK)], xbuf[u],
                                      sem_g[u]).wait()
                pltpu.make_async_copy(e_hbm.at[pl.ds(0, _CHUNK)], ebuf[u],
                                      sem_e[u]).wait()

                hi_mask = jnp.full((_LANES,), -65536, jnp.int32)

                def row_body(r, _):
                    for jj in range(_DP // _LANES):
                        s16 = pl.ds(jj * _LANES, _LANES)
                        xv = xbuf[u][r, s16]
                        ev = ebuf[u][r, s16]
                        bc = lax.bitcast_convert_type
                        lo = (bc(xv << 16, jnp.float32)
                              + bc(ev << 16, jnp.float32))
                        hi = (bc(xv & hi_mask, jnp.float32)
                              + bc(ev & hi_mask, jnp.float32))
                        mbuf[u][r, s16] = jnp.maximum(lo, 0.0)
                        mbuf[u][r, pl.ds(_DP + jj * _LANES, _LANES)] = (
                            jnp.maximum(hi, 0.0))
                    return 0

                lax.fori_loop(0, _CHUNK, row_body, 0)
                pltpu.make_async_copy(dst_hbm.at[pl.ds(0, _CHUNK)], dst_v[u],
                                      sem_id[u]).wait()
                pltpu.async_copy(mbuf[u], acc.at[dst_v[u]], sem_sc[u],
                                 add=True)
        return 0

    lax.fori_loop(0, (_NCH // _NW + _NSLOT) // _NSLOT, tri_body, 0)

    for u in range(_NSLOT):
        pltpu.make_async_copy(mbuf[u], acc.at[pl.ds(0, _CHUNK)],
                              sem_sc[u]).wait()

    plsc.subcore_barrier()

    row0 = sid * _STRIPE
    pltpu.sync_copy(acc.at[pl.ds(row0, _STRIPE)],
                    out_hbm.at[pl.ds(cid * _N + row0, _STRIPE)])

    @pl.when(sid == 15)
    def _write_tail():
        pltpu.sync_copy(acc.at[pl.ds(16 * _STRIPE, _TAIL)],
                        out_hbm.at[pl.ds(cid * _N + 16 * _STRIPE, _TAIL)])


def _sc_edge(x, e_l, src, dst):
    mesh = plsc.VectorSubcoreMesh(core_axis_name="c", subcore_axis_name="s")
    f = functools.partial(
        pl.kernel,
        out_type=jax.ShapeDtypeStruct((2 * _N, _D), jnp.float32),
        mesh=mesh,
        compiler_params=pltpu.CompilerParams(use_tc_tiling_on_sc=False),
        scratch_types=(
            [pltpu.VMEM((_CHUNK,), jnp.int32)] * 3
            + [pltpu.VMEM((_CHUNK,), jnp.int32)] * 3
            + [pltpu.VMEM((_CHUNK, _DP), jnp.int32)] * 3
            + [pltpu.VMEM((_CHUNK, _DP), jnp.int32)] * 3
            + [pltpu.VMEM((_CHUNK, _D), jnp.float32)] * 3
            + [pltpu.VMEM_SHARED((_N, _D), jnp.float32)]
            + [pltpu.SemaphoreType.DMA] * 15
        ),
    )(_sc_edge_body)
    return f(x, e_l, src, dst)


def _node_body(x_ref, agg_ref, w_ref, b_ref, g_ref, bt_ref, o_ref, osc_ref):
    xa = x_ref[...] + agg_ref[0:_N, :] + agg_ref[_N:2 * _N, :]
    h = jnp.dot(xa, w_ref[...], preferred_element_type=jnp.float32) + b_ref[...]
    mu = jnp.mean(h, axis=0, keepdims=True)
    var = jnp.mean(h * h, axis=0, keepdims=True) - mu * mu
    hn = (h - mu) * lax.rsqrt(var + 1e-5) * g_ref[...] + bt_ref[...]
    hl = jnp.where(hn >= 0, hn, 0.01 * hn)
    o_ref[...] = hl
    osc_ref[...] = _pack_rows(hl)


def _node(x, agg2, w, b, g, bt):
    return pl.pallas_call(
        _node_body,
        out_shape=[jax.ShapeDtypeStruct((_N, _D), jnp.float32),
                   jax.ShapeDtypeStruct((_N, _DP), jnp.int32)],
    )(x, agg2, w, b.reshape(1, _D), g.reshape(1, _D), bt.reshape(1, _D))


def _final_body(h1_ref, h2_ref, h3_ref, brow_ref, bcol_ref, wl_ref, bl_ref,
                o_ref):
    w1 = wl_ref[0:_D, :]
    w2 = wl_ref[_D:2 * _D, :]
    w3 = wl_ref[2 * _D:3 * _D, :]
    w4 = wl_ref[3 * _D:4 * _D, :]
    h3 = h3_ref[...]
    t = (jnp.dot(h1_ref[...], w1, preferred_element_type=jnp.float32)
         + jnp.dot(h2_ref[...], w2, preferred_element_type=jnp.float32)
         + jnp.dot(h3, w3, preferred_element_type=jnp.float32))
    oh = (lax.broadcasted_iota(jnp.int32, (_G, _N), 0)
          == brow_ref[...]).astype(jnp.float32)
    pool = jnp.dot(oh, h3, preferred_element_type=jnp.float32)
    s = jnp.dot(pool, w4, preferred_element_type=jnp.float32)
    oht = (lax.broadcasted_iota(jnp.int32, (_N, _G), 1)
           == bcol_ref[...]).astype(jnp.float32)
    pooled = jnp.dot(oht, s, preferred_element_type=jnp.float32)
    z = t + pooled + bl_ref[...]
    o_ref[...] = 1.0 / (1.0 + jnp.exp(-z))


def _final(h1, h2, h3, batch, wl, bl):
    return pl.pallas_call(
        _final_body,
        out_shape=jax.ShapeDtypeStruct((_N, 1), jnp.float32),
    )(h1, h2, h3, batch.reshape(1, _N), batch.reshape(_N, 1), wl,
      bl.reshape(1, 1))


def kernel(x, edge_index, edge_attr, batch,
           W1e, b1e, W1, b1, g1, bt1,
           W2e, b2e, W2, b2, g2, bt2,
           W3e, b3e, W3, b3, g3, bt3,
           Wl, bl):
    src = edge_index[0]
    dst = edge_index[1]
    e1, e2, e3 = _edge_mlp(edge_attr, W1e, b1e, W2e, b2e, W3e, b3e)

    h = x
    h_sc = _xpack(x)
    hs = []
    for e_l, w, b, g, bt in ((e1, W1, b1, g1, bt1),
                             (e2, W2, b2, g2, bt2),
                             (e3, W3, b3, g3, bt3)):
        agg2 = _sc_edge(h_sc, e_l, src, dst)
        h, h_sc = _node(h, agg2, w, b, g, bt)
        hs.append(h)

    return _final(hs[0], hs[1], hs[2], batch, Wl, bl)


# speedup vs baseline: 1.1441x; 1.1441x over previous
import functools

import jax
import jax.numpy as jnp
from jax import lax
from jax.experimental import pallas as pl
from jax.experimental.pallas import tpu as pltpu
from jax.experimental.pallas import tpu_sc as plsc

_N = 10000
_D = 128
_E = 320000
_G = 64
_LANES = 16
_CHUNK = 64
_EPR = _CHUNK // 2
_NCH = _E // _CHUNK
_NSLOT = 3
_NW = 32
_STRIPE = 624
_TAIL = _N - 16 * _STRIPE

_DP = _D // 2


def _pack_rows(h):
    bits = lax.bitcast_convert_type(h, jnp.int32)
    r = (bits + 32768) >> 16
    return (r[:, 0:_DP] & 65535) | (r[:, _DP:_D] << 16)


def _edge_mlp_body(ea_ref, w1_ref, b1_ref, w2_ref, b2_ref, w3_ref, b3_ref,
                   o1_ref, o2_ref, o3_ref):
    ea = ea_ref[...]
    ed = ea.shape[1] // 2
    ea_a = ea[:, 0:ed]
    ea_b = ea[:, ed:2 * ed]

    def pair(w_ref, b_ref):
        w = w_ref[...]
        b = b_ref[...]
        pa = _pack_rows(jnp.dot(ea_a, w, preferred_element_type=jnp.float32)
                        + b)
        pb = _pack_rows(jnp.dot(ea_b, w, preferred_element_type=jnp.float32)
                        + b)
        return jnp.concatenate([pa, pb], axis=1)

    o1_ref[...] = pair(w1_ref, b1_ref)
    o2_ref[...] = pair(w2_ref, b2_ref)
    o3_ref[...] = pair(w3_ref, b3_ref)


def _edge_mlp(edge_attr2, w1, b1, w2, b2, w3, b3):
    blk = 1000
    grid = (_E // 2) // blk
    ed2 = edge_attr2.shape[1]
    out_spec = pl.BlockSpec((blk, _D), lambda i: (i, 0))
    w_spec = pl.BlockSpec((ed2 // 2, _D), lambda i: (0, 0))
    b_spec = pl.BlockSpec((1, _D), lambda i: (0, 0))
    return pl.pallas_call(
        _edge_mlp_body,
        grid=(grid,),
        in_specs=[pl.BlockSpec((blk, ed2), lambda i: (i, 0)),
                  w_spec, b_spec, w_spec, b_spec, w_spec, b_spec],
        out_specs=[out_spec, out_spec, out_spec],
        out_shape=[jax.ShapeDtypeStruct((_E // 2, _D), jnp.int32)] * 3,
    )(edge_attr2, w1, b1.reshape(1, _D), w2, b2.reshape(1, _D),
      w3, b3.reshape(1, _D))


def _sc_edge_body(x_hbm, e_hbm, src_hbm, dst_hbm, out_hbm, *sc):
    src_v = sc[0:3]
    dst_v = sc[3:6]
    xbuf = sc[6:9]
    ebuf = sc[9:12]
    acc = sc[12]
    sem_is = sc[13:16]
    sem_id = sc[16:19]
    sem_g = sc[19:22]
    sem_e = sc[22:25]
    sem_sc = sc[25:28]

    cid = lax.axis_index("c")
    sid = lax.axis_index("s")
    wid = sid * 2 + cid

    def zero_row(r, _):
        for j in range(_D // _LANES):
            xbuf[0][r, pl.ds(j * _LANES, _LANES)] = jnp.zeros(
                (_LANES,), jnp.float32)
        return 0

    lax.fori_loop(0, _CHUNK, zero_row, 0)
    nfull = _STRIPE // _CHUNK
    for k in range(nfull):
        pltpu.sync_copy(xbuf[0], acc.at[pl.ds(sid * _STRIPE + k * _CHUNK,
                                              _CHUNK)])
    pltpu.sync_copy(xbuf[0].at[pl.ds(0, _STRIPE - nfull * _CHUNK)],
                    acc.at[pl.ds(sid * _STRIPE + nfull * _CHUNK,
                                 _STRIPE - nfull * _CHUNK)])

    @pl.when(sid == 15)
    def _zero_tail():
        pltpu.sync_copy(xbuf[0].at[pl.ds(0, _TAIL)],
                        acc.at[pl.ds(16 * _STRIPE, _TAIL)])

    plsc.subcore_barrier()

    nbase = _NCH // _NW
    nch = nbase + jnp.where(wid < _NCH - nbase * _NW, 1, 0)

    def issue_loads(k, slot):
        c = k * _NW + wid
        base = c * _CHUNK
        pltpu.async_copy(src_hbm.at[pl.ds(base, _CHUNK)], src_v[slot],
                         sem_is[slot])
        pltpu.async_copy(dst_hbm.at[pl.ds(base, _CHUNK)], dst_v[slot],
                         sem_id[slot])
        pltpu.async_copy(e_hbm.at[pl.ds(c * _EPR, _EPR)], ebuf[slot],
                         sem_e[slot])

    def issue_gather(slot):
        pltpu.make_async_copy(src_hbm.at[pl.ds(0, _CHUNK)], src_v[slot],
                              sem_is[slot]).wait()
        pltpu.async_copy(x_hbm.at[src_v[slot]], xbuf[slot], sem_g[slot])

    issue_loads(jnp.int32(0), 0)
    issue_loads(jnp.int32(1), 1)
    issue_gather(0)

    def tri_body(i3, _):
        for u in range(_NSLOT):
            k = i3 * _NSLOT + u
            s1 = (u + 1) % _NSLOT
            sj = (u + 2) % _NSLOT
            j = k + 2

            @pl.when((k >= 1) & (j < nch))
            def _refill_wait():
                pltpu.make_async_copy(xbuf[sj], acc.at[pl.ds(0, _CHUNK)],
                                      sem_sc[sj]).wait()
                issue_loads(j, sj)

            @pl.when((k == 0) & (j < nch))
            def _refill_fresh():
                issue_loads(j, sj)

            @pl.when(k + 1 < nch)
            def _gather_next():
                issue_gather(s1)

            @pl.when(k < nch)
            def _process():
                pltpu.make_async_copy(x_hbm.at[pl.ds(0, _CHUNK)], xbuf[u],
                                      sem_g[u]).wait()
                pltpu.make_async_copy(e_hbm.at[pl.ds(0, _EPR)], ebuf[u],
                                      sem_e[u]).wait()

                hi_mask = jnp.full((_LANES,), -65536, jnp.int32)
                bc = lax.bitcast_convert_type

                def row_body(pr, _):
                    for half in range(2):
                        r = 2 * pr + half
                        for jj in range(_DP // _LANES):
                            se = pl.ds(half * _DP + jj * _LANES, _LANES)
                            slo = pl.ds(jj * _LANES, _LANES)
                            shi = pl.ds(_DP + jj * _LANES, _LANES)
                            ev = ebuf[u][pr, se]
                            lo = xbuf[u][r, slo] + bc(ev << 16, jnp.float32)
                            hi = xbuf[u][r, shi] + bc(ev & hi_mask,
                                                      jnp.float32)
                            xbuf[u][r, slo] = jnp.maximum(lo, 0.0)
                            xbuf[u][r, shi] = jnp.maximum(hi, 0.0)
                    return 0

                lax.fori_loop(0, _EPR, row_body, 0)
                pltpu.make_async_copy(dst_hbm.at[pl.ds(0, _CHUNK)], dst_v[u],
                                      sem_id[u]).wait()
                pltpu.async_copy(xbuf[u], acc.at[dst_v[u]], sem_sc[u],
                                 add=True)
        return 0

    lax.fori_loop(0, (_NCH // _NW + _NSLOT) // _NSLOT, tri_body, 0)

    for u in range(_NSLOT):
        pltpu.make_async_copy(xbuf[u], acc.at[pl.ds(0, _CHUNK)],
                              sem_sc[u]).wait()

    plsc.subcore_barrier()

    row0 = sid * _STRIPE
    pltpu.sync_copy(acc.at[pl.ds(row0, _STRIPE)],
                    out_hbm.at[pl.ds(cid * _N + row0, _STRIPE)])

    @pl.when(sid == 15)
    def _write_tail():
        pltpu.sync_copy(acc.at[pl.ds(16 * _STRIPE, _TAIL)],
                        out_hbm.at[pl.ds(cid * _N + 16 * _STRIPE, _TAIL)])


def _sc_edge(x, e_l, src, dst):
    mesh = plsc.VectorSubcoreMesh(core_axis_name="c", subcore_axis_name="s")
    f = functools.partial(
        pl.kernel,
        out_type=jax.ShapeDtypeStruct((2 * _N, _D), jnp.float32),
        mesh=mesh,
        scratch_types=(
            [pltpu.VMEM((_CHUNK,), jnp.int32)] * 3
            + [pltpu.VMEM((_CHUNK,), jnp.int32)] * 3
            + [pltpu.VMEM((_CHUNK, _D), jnp.float32)] * 3
            + [pltpu.VMEM((_EPR, _D), jnp.int32)] * 3
            + [pltpu.VMEM_SHARED((_N, _D), jnp.float32)]
            + [pltpu.SemaphoreType.DMA] * 15
        ),
    )(_sc_edge_body)
    return f(x, e_l, src, dst)


def _node_body(x_ref, agg_ref, w_ref, b_ref, g_ref, bt_ref, o_ref):
    xa = x_ref[...] + agg_ref[0:_N, :] + agg_ref[_N:2 * _N, :]
    h = jnp.dot(xa, w_ref[...], preferred_element_type=jnp.float32) + b_ref[...]
    mu = jnp.mean(h, axis=0, keepdims=True)
    var = jnp.mean(h * h, axis=0, keepdims=True) - mu * mu
    hn = (h - mu) * lax.rsqrt(var + 1e-5) * g_ref[...] + bt_ref[...]
    o_ref[...] = jnp.where(hn >= 0, hn, 0.01 * hn)


def _node(x, agg2, w, b, g, bt):
    return pl.pallas_call(
        _node_body,
        out_shape=jax.ShapeDtypeStruct((_N, _D), jnp.float32),
    )(x, agg2, w, b.reshape(1, _D), g.reshape(1, _D), bt.reshape(1, _D))


def _final_body(h1_ref, h2_ref, h3_ref, brow_ref, bcol_ref, wl_ref, bl_ref,
                o_ref):
    w1 = wl_ref[0:_D, :]
    w2 = wl_ref[_D:2 * _D, :]
    w3 = wl_ref[2 * _D:3 * _D, :]
    w4 = wl_ref[3 * _D:4 * _D, :]
    h3 = h3_ref[...]
    t = (jnp.dot(h1_ref[...], w1, preferred_element_type=jnp.float32)
         + jnp.dot(h2_ref[...], w2, preferred_element_type=jnp.float32)
         + jnp.dot(h3, w3, preferred_element_type=jnp.float32))
    oh = (lax.broadcasted_iota(jnp.int32, (_G, _N), 0)
          == brow_ref[...]).astype(jnp.float32)
    pool = jnp.dot(oh, h3, preferred_element_type=jnp.float32)
    s = jnp.dot(pool, w4, preferred_element_type=jnp.float32)
    oht = (lax.broadcasted_iota(jnp.int32, (_N, _G), 1)
           == bcol_ref[...]).astype(jnp.float32)
    pooled = jnp.dot(oht, s, preferred_element_type=jnp.float32)
    z = t + pooled + bl_ref[...]
    o_ref[...] = 1.0 / (1.0 + jnp.exp(-z))


def _final(h1, h2, h3, batch, wl, bl):
    return pl.pallas_call(
        _final_body,
        out_shape=jax.ShapeDtypeStruct((_N, 1), jnp.float32),
    )(h1, h2, h3, batch.reshape(1, _N), batch.reshape(_N, 1), wl,
      bl.reshape(1, 1))


def kernel(x, edge_index, edge_attr, batch,
           W1e, b1e, W1, b1, g1, bt1,
           W2e, b2e, W2, b2, g2, bt2,
           W3e, b3e, W3, b3, g3, bt3,
           Wl, bl):
    src = edge_index[0]
    dst = edge_index[1]
    ed = edge_attr.shape[1]
    e1, e2, e3 = _edge_mlp(edge_attr.reshape(_E // 2, 2 * ed),
                           W1e, b1e, W2e, b2e, W3e, b3e)

    h = x
    hs = []
    for e_l, w, b, g, bt in ((e1, W1, b1, g1, bt1),
                             (e2, W2, b2, g2, bt2),
                             (e3, W3, b3, g3, bt3)):
        agg2 = _sc_edge(h, e_l, src, dst)
        h = _node(h, agg2, w, b, g, bt)
        hs.append(h)

    return _final(hs[0], hs[1], hs[2], batch, Wl, bl)


# speedup vs baseline: 1.6483x; 1.4407x over previous
import functools

import jax
import jax.numpy as jnp
from jax import lax
from jax.experimental import pallas as pl
from jax.experimental.pallas import tpu as pltpu
from jax.experimental.pallas import tpu_sc as plsc

_N = 10000
_D = 128
_E = 320000
_G = 64
_LANES = 16
_CHUNK = 64
_NCH = _E // _CHUNK
_NSLOT = 3
_NW = 32
_STRIPE = 624
_TAIL = _N - 16 * _STRIPE


def _edge_mlp_body(ea_ref, w1_ref, b1_ref, w2_ref, b2_ref, w3_ref, b3_ref,
                   o1_ref, o2_ref, o3_ref):
    ea = ea_ref[...]
    o1_ref[...] = jnp.dot(ea, w1_ref[...],
                          preferred_element_type=jnp.float32) + b1_ref[...]
    o2_ref[...] = jnp.dot(ea, w2_ref[...],
                          preferred_element_type=jnp.float32) + b2_ref[...]
    o3_ref[...] = jnp.dot(ea, w3_ref[...],
                          preferred_element_type=jnp.float32) + b3_ref[...]


def _edge_mlp(edge_attr, w1, b1, w2, b2, w3, b3):
    blk = 2000
    grid = _E // blk
    ed = edge_attr.shape[1]
    out_spec = pl.BlockSpec((blk, _D), lambda i: (i, 0))
    w_spec = pl.BlockSpec((ed, _D), lambda i: (0, 0))
    b_spec = pl.BlockSpec((1, _D), lambda i: (0, 0))
    return pl.pallas_call(
        _edge_mlp_body,
        grid=(grid,),
        in_specs=[pl.BlockSpec((blk, ed), lambda i: (i, 0)),
                  w_spec, b_spec, w_spec, b_spec, w_spec, b_spec],
        out_specs=[out_spec, out_spec, out_spec],
        out_shape=[jax.ShapeDtypeStruct((_E, _D), jnp.float32)] * 3,
    )(edge_attr, w1, b1.reshape(1, _D), w2, b2.reshape(1, _D),
      w3, b3.reshape(1, _D))


def _sc_edge_body(x_hbm, e_hbm, src_hbm, dst_hbm, out_hbm, *sc):
    src_v = sc[0:3]
    dst_v = sc[3:6]
    xbuf = sc[6:9]
    ebuf = sc[9:12]
    acc = sc[12]
    sem_is = sc[13:16]
    sem_id = sc[16:19]
    sem_g = sc[19:22]
    sem_e = sc[22:25]
    sem_sc = sc[25:28]

    cid = lax.axis_index("c")
    sid = lax.axis_index("s")
    wid = sid * 2 + cid

    def zero_row(r, _):
        for j in range(_D // _LANES):
            xbuf[0][r, pl.ds(j * _LANES, _LANES)] = jnp.zeros(
                (_LANES,), jnp.float32)
        return 0

    lax.fori_loop(0, _CHUNK, zero_row, 0)
    nfull = _STRIPE // _CHUNK
    for k in range(nfull):
        pltpu.sync_copy(xbuf[0], acc.at[pl.ds(sid * _STRIPE + k * _CHUNK,
                                              _CHUNK)])
    pltpu.sync_copy(xbuf[0].at[pl.ds(0, _STRIPE - nfull * _CHUNK)],
                    acc.at[pl.ds(sid * _STRIPE + nfull * _CHUNK,
                                 _STRIPE - nfull * _CHUNK)])

    @pl.when(sid == 15)
    def _zero_tail():
        pltpu.sync_copy(xbuf[0].at[pl.ds(0, _TAIL)],
                        acc.at[pl.ds(16 * _STRIPE, _TAIL)])

    plsc.subcore_barrier()

    nbase = _NCH // _NW
    nch = nbase + jnp.where(wid < _NCH - nbase * _NW, 1, 0)

    def issue_loads(k, slot):
        c = k * _NW + wid
        base = c * _CHUNK
        pltpu.async_copy(src_hbm.at[pl.ds(base, _CHUNK)], src_v[slot],
                         sem_is[slot])
        pltpu.async_copy(dst_hbm.at[pl.ds(base, _CHUNK)], dst_v[slot],
                         sem_id[slot])
        pltpu.async_copy(e_hbm.at[pl.ds(base, _CHUNK)], ebuf[slot],
                         sem_e[slot])

    def issue_gather(slot):
        pltpu.make_async_copy(src_hbm.at[pl.ds(0, _CHUNK)], src_v[slot],
                              sem_is[slot]).wait()
        pltpu.async_copy(x_hbm.at[src_v[slot]], xbuf[slot], sem_g[slot])

    issue_loads(jnp.int32(0), 0)
    issue_loads(jnp.int32(1), 1)
    issue_gather(0)

    def tri_body(i3, _):
        for u in range(_NSLOT):
            k = i3 * _NSLOT + u
            s1 = (u + 1) % _NSLOT
            sj = (u + 2) % _NSLOT
            j = k + 2

            @pl.when((k >= 1) & (j < nch))
            def _refill_wait():
                pltpu.make_async_copy(ebuf[sj], acc.at[pl.ds(0, _CHUNK)],
                                      sem_sc[sj]).wait()
                issue_loads(j, sj)

            @pl.when((k == 0) & (j < nch))
            def _refill_fresh():
                issue_loads(j, sj)

            @pl.when(k + 1 < nch)
            def _gather_next():
                issue_gather(s1)

            @pl.when(k < nch)
            def _process():
                pltpu.make_async_copy(x_hbm.at[pl.ds(0, _CHUNK)], xbuf[u],
                                      sem_g[u]).wait()
                pltpu.make_async_copy(e_hbm.at[pl.ds(0, _CHUNK)], ebuf[u],
                                      sem_e[u]).wait()

                def row_body(r, _):
                    for jj in range(_D // _LANES):
                        s = pl.ds(jj * _LANES, _LANES)
                        ebuf[u][r, s] = jnp.maximum(
                            ebuf[u][r, s] + xbuf[u][r, s], 0.0)
                    return 0

                lax.fori_loop(0, _CHUNK, row_body, 0)
                pltpu.make_async_copy(dst_hbm.at[pl.ds(0, _CHUNK)], dst_v[u],
                                      sem_id[u]).wait()
                pltpu.async_copy(ebuf[u], acc.at[dst_v[u]], sem_sc[u],
                                 add=True)
        return 0

    lax.fori_loop(0, (_NCH // _NW + _NSLOT) // _NSLOT, tri_body, 0)

    for u in range(_NSLOT):
        pltpu.make_async_copy(ebuf[u], acc.at[pl.ds(0, _CHUNK)],
                              sem_sc[u]).wait()

    plsc.subcore_barrier()

    row0 = sid * _STRIPE
    pltpu.sync_copy(acc.at[pl.ds(row0, _STRIPE)],
                    out_hbm.at[pl.ds(cid * _N + row0, _STRIPE)])

    @pl.when(sid == 15)
    def _write_tail():
        pltpu.sync_copy(acc.at[pl.ds(16 * _STRIPE, _TAIL)],
                        out_hbm.at[pl.ds(cid * _N + 16 * _STRIPE, _TAIL)])


def _sc_edge(x, e_l, src, dst):
    mesh = plsc.VectorSubcoreMesh(core_axis_name="c", subcore_axis_name="s")
    f = functools.partial(
        pl.kernel,
        out_type=jax.ShapeDtypeStruct((2 * _N, _D), jnp.float32),
        mesh=mesh,
        scratch_types=(
            [pltpu.VMEM((_CHUNK,), jnp.int32)] * 3
            + [pltpu.VMEM((_CHUNK,), jnp.int32)] * 3
            + [pltpu.VMEM((_CHUNK, _D), jnp.float32)] * 3
            + [pltpu.VMEM((_CHUNK, _D), jnp.float32)] * 3
            + [pltpu.VMEM_SHARED((_N, _D), jnp.float32)]
            + [pltpu.SemaphoreType.DMA] * 15
        ),
    )(_sc_edge_body)
    return f(x, e_l, src, dst)


def _node_body(x_ref, agg_ref, w_ref, b_ref, g_ref, bt_ref, o_ref):
    xa = x_ref[...] + agg_ref[0:_N, :] + agg_ref[_N:2 * _N, :]
    h = jnp.dot(xa, w_ref[...], preferred_element_type=jnp.float32) + b_ref[...]
    mu = jnp.mean(h, axis=0, keepdims=True)
    var = jnp.mean(h * h, axis=0, keepdims=True) - mu * mu
    hn = (h - mu) * lax.rsqrt(var + 1e-5) * g_ref[...] + bt_ref[...]
    o_ref[...] = jnp.where(hn >= 0, hn, 0.01 * hn)


def _node(x, agg2, w, b, g, bt):
    return pl.pallas_call(
        _node_body,
        out_shape=jax.ShapeDtypeStruct((_N, _D), jnp.float32),
    )(x, agg2, w, b.reshape(1, _D), g.reshape(1, _D), bt.reshape(1, _D))


def _final_body(h1_ref, h2_ref, h3_ref, brow_ref, bcol_ref, wl_ref, bl_ref,
                o_ref):
    w1 = wl_ref[0:_D, :]
    w2 = wl_ref[_D:2 * _D, :]
    w3 = wl_ref[2 * _D:3 * _D, :]
    w4 = wl_ref[3 * _D:4 * _D, :]
    h3 = h3_ref[...]
    t = (jnp.dot(h1_ref[...], w1, preferred_element_type=jnp.float32)
         + jnp.dot(h2_ref[...], w2, preferred_element_type=jnp.float32)
         + jnp.dot(h3, w3, preferred_element_type=jnp.float32))
    oh = (lax.broadcasted_iota(jnp.int32, (_G, _N), 0)
          == brow_ref[...]).astype(jnp.float32)
    pool = jnp.dot(oh, h3, preferred_element_type=jnp.float32)
    s = jnp.dot(pool, w4, preferred_element_type=jnp.float32)
    oht = (lax.broadcasted_iota(jnp.int32, (_N, _G), 1)
           == bcol_ref[...]).astype(jnp.float32)
    pooled = jnp.dot(oht, s, preferred_element_type=jnp.float32)
    z = t + pooled + bl_ref[...]
    o_ref[...] = 1.0 / (1.0 + jnp.exp(-z))


def _final(h1, h2, h3, batch, wl, bl):
    return pl.pallas_call(
        _final_body,
        out_shape=jax.ShapeDtypeStruct((_N, 1), jnp.float32),
    )(h1, h2, h3, batch.reshape(1, _N), batch.reshape(_N, 1), wl,
      bl.reshape(1, 1))


def kernel(x, edge_index, edge_attr, batch,
           W1e, b1e, W1, b1, g1, bt1,
           W2e, b2e, W2, b2, g2, bt2,
           W3e, b3e, W3, b3, g3, bt3,
           Wl, bl):
    src = edge_index[0]
    dst = edge_index[1]
    e1, e2, e3 = _edge_mlp(edge_attr, W1e, b1e, W2e, b2e, W3e, b3e)

    h = x
    hs = []
    for e_l, w, b, g, bt in ((e1, W1, b1, g1, bt1),
                             (e2, W2, b2, g2, bt2),
                             (e3, W3, b3, g3, bt3)):
        agg2 = _sc_edge(h, e_l, src, dst)
        h = _node(h, agg2, w, b, g, bt)
        hs.append(h)

    return _final(hs[0], hs[1], hs[2], batch, Wl, bl)


# speedup vs baseline: 1.6504x; 1.0013x over previous
import functools

import jax
import jax.numpy as jnp
from jax import lax
from jax.experimental import pallas as pl
from jax.experimental.pallas import tpu as pltpu
from jax.experimental.pallas import tpu_sc as plsc

_N = 10000
_D = 128
_E = 320000
_G = 64
_LANES = 16
_CHUNK = 64
_NCH = _E // _CHUNK
_NSLOT = 3
_NW = 32
_STRIPE = 624
_TAIL = _N - 16 * _STRIPE


def _edge_mlp_body(ea_ref, w1_ref, b1_ref, w2_ref, b2_ref, w3_ref, b3_ref,
                   o1_ref, o2_ref, o3_ref):
    ea = ea_ref[...]
    o1_ref[...] = jnp.dot(ea, w1_ref[...],
                          preferred_element_type=jnp.float32) + b1_ref[...]
    o2_ref[...] = jnp.dot(ea, w2_ref[...],
                          preferred_element_type=jnp.float32) + b2_ref[...]
    o3_ref[...] = jnp.dot(ea, w3_ref[...],
                          preferred_element_type=jnp.float32) + b3_ref[...]


def _edge_mlp(edge_attr, w1, b1, w2, b2, w3, b3):
    blk = 2000
    grid = _E // blk
    ed = edge_attr.shape[1]
    out_spec = pl.BlockSpec((blk, _D), lambda i: (i, 0))
    w_spec = pl.BlockSpec((ed, _D), lambda i: (0, 0))
    b_spec = pl.BlockSpec((1, _D), lambda i: (0, 0))
    return pl.pallas_call(
        _edge_mlp_body,
        grid=(grid,),
        in_specs=[pl.BlockSpec((blk, ed), lambda i: (i, 0)),
                  w_spec, b_spec, w_spec, b_spec, w_spec, b_spec],
        out_specs=[out_spec, out_spec, out_spec],
        out_shape=[jax.ShapeDtypeStruct((_E, _D), jnp.float32)] * 3,
    )(edge_attr, w1, b1.reshape(1, _D), w2, b2.reshape(1, _D),
      w3, b3.reshape(1, _D))


def _sc_edge_body(x_hbm, e_hbm, src_hbm, dst_hbm, out_hbm, *sc):
    src_v = sc[0:3]
    dst_v = sc[3:6]
    xbuf = sc[6:9]
    ebuf = sc[9:12]
    acc = sc[12]
    sem_is = sc[13:16]
    sem_id = sc[16:19]
    sem_g = sc[19:22]
    sem_e = sc[22:25]
    sem_sc = sc[25:28]

    cid = lax.axis_index("c")
    sid = lax.axis_index("s")
    wid = sid * 2 + cid

    def zero_row(r, _):
        for j in range(_D // _LANES):
            xbuf[0][r, pl.ds(j * _LANES, _LANES)] = jnp.zeros(
                (_LANES,), jnp.float32)
        return 0

    lax.fori_loop(0, _CHUNK, zero_row, 0)
    nfull = _STRIPE // _CHUNK
    for k in range(nfull):
        pltpu.sync_copy(xbuf[0], acc.at[pl.ds(sid * _STRIPE + k * _CHUNK,
                                              _CHUNK)])
    pltpu.sync_copy(xbuf[0].at[pl.ds(0, _STRIPE - nfull * _CHUNK)],
                    acc.at[pl.ds(sid * _STRIPE + nfull * _CHUNK,
                                 _STRIPE - nfull * _CHUNK)])

    @pl.when(sid == 15)
    def _zero_tail():
        pltpu.sync_copy(xbuf[0].at[pl.ds(0, _TAIL)],
                        acc.at[pl.ds(16 * _STRIPE, _TAIL)])

    plsc.subcore_barrier()

    nbase = _NCH // _NW
    nch = nbase + jnp.where(wid < _NCH - nbase * _NW, 1, 0)

    def issue_loads(k, slot):
        c = k * _NW + wid
        base = c * _CHUNK
        pltpu.async_copy(src_hbm.at[pl.ds(base, _CHUNK)], src_v[slot],
                         sem_is[slot])
        pltpu.async_copy(dst_hbm.at[pl.ds(base, _CHUNK)], dst_v[slot],
                         sem_id[slot])
        pltpu.async_copy(e_hbm.at[pl.ds(base, _CHUNK)], ebuf[slot],
                         sem_e[slot])

    def issue_gather(slot):
        pltpu.make_async_copy(src_hbm.at[pl.ds(0, _CHUNK)], src_v[slot],
                              sem_is[slot]).wait()
        pltpu.async_copy(x_hbm.at[src_v[slot]], xbuf[slot], sem_g[slot])

    issue_loads(jnp.int32(0), 0)
    issue_loads(jnp.int32(1), 1)
    issue_gather(0)

    def tri_body(i3, _):
        for u in range(_NSLOT):
            k = i3 * _NSLOT + u
            s1 = (u + 1) % _NSLOT
            sj = (u + 2) % _NSLOT
            j = k + 2

            @pl.when((k >= 1) & (j < nch))
            def _refill_wait():
                pltpu.make_async_copy(ebuf[sj], acc.at[pl.ds(0, _CHUNK)],
                                      sem_sc[sj]).wait()
                issue_loads(j, sj)

            @pl.when((k == 0) & (j < nch))
            def _refill_fresh():
                issue_loads(j, sj)

            @pl.when(k + 1 < nch)
            def _gather_next():
                issue_gather(s1)

            @pl.when(k < nch)
            def _process():
                pltpu.make_async_copy(x_hbm.at[pl.ds(0, _CHUNK)], xbuf[u],
                                      sem_g[u]).wait()
                pltpu.make_async_copy(e_hbm.at[pl.ds(0, _CHUNK)], ebuf[u],
                                      sem_e[u]).wait()

                @plsc.parallel_loop(0, _CHUNK, step=1)
                def _rows(r):
                    for jj in range(_D // _LANES):
                        s = pl.ds(jj * _LANES, _LANES)
                        ebuf[u][r, s] = jnp.maximum(
                            ebuf[u][r, s] + xbuf[u][r, s], 0.0)
                pltpu.make_async_copy(dst_hbm.at[pl.ds(0, _CHUNK)], dst_v[u],
                                      sem_id[u]).wait()
                pltpu.async_copy(ebuf[u], acc.at[dst_v[u]], sem_sc[u],
                                 add=True)
        return 0

    lax.fori_loop(0, (_NCH // _NW + _NSLOT) // _NSLOT, tri_body, 0)

    for u in range(_NSLOT):
        pltpu.make_async_copy(ebuf[u], acc.at[pl.ds(0, _CHUNK)],
                              sem_sc[u]).wait()

    plsc.subcore_barrier()

    row0 = sid * _STRIPE
    pltpu.sync_copy(acc.at[pl.ds(row0, _STRIPE)],
                    out_hbm.at[pl.ds(cid * _N + row0, _STRIPE)])

    @pl.when(sid == 15)
    def _write_tail():
        pltpu.sync_copy(acc.at[pl.ds(16 * _STRIPE, _TAIL)],
                        out_hbm.at[pl.ds(cid * _N + 16 * _STRIPE, _TAIL)])


def _sc_edge(x, e_l, src, dst):
    mesh = plsc.VectorSubcoreMesh(core_axis_name="c", subcore_axis_name="s")
    f = functools.partial(
        pl.kernel,
        out_type=jax.ShapeDtypeStruct((2 * _N, _D), jnp.float32),
        mesh=mesh,
        scratch_types=(
            [pltpu.VMEM((_CHUNK,), jnp.int32)] * 3
            + [pltpu.VMEM((_CHUNK,), jnp.int32)] * 3
            + [pltpu.VMEM((_CHUNK, _D), jnp.float32)] * 3
            + [pltpu.VMEM((_CHUNK, _D), jnp.float32)] * 3
            + [pltpu.VMEM_SHARED((_N, _D), jnp.float32)]
            + [pltpu.SemaphoreType.DMA] * 15
        ),
    )(_sc_edge_body)
    return f(x, e_l, src, dst)


def _node_body(x_ref, agg_ref, w_ref, b_ref, g_ref, bt_ref, o_ref):
    xa = x_ref[...] + agg_ref[0:_N, :] + agg_ref[_N:2 * _N, :]
    h = jnp.dot(xa, w_ref[...], preferred_element_type=jnp.float32) + b_ref[...]
    mu = jnp.mean(h, axis=0, keepdims=True)
    var = jnp.mean(h * h, axis=0, keepdims=True) - mu * mu
    hn = (h - mu) * lax.rsqrt(var + 1e-5) * g_ref[...] + bt_ref[...]
    o_ref[...] = jnp.where(hn >= 0, hn, 0.01 * hn)


def _node(x, agg2, w, b, g, bt):
    return pl.pallas_call(
        _node_body,
        out_shape=jax.ShapeDtypeStruct((_N, _D), jnp.float32),
    )(x, agg2, w, b.reshape(1, _D), g.reshape(1, _D), bt.reshape(1, _D))


def _final_body(h1_ref, h2_ref, h3_ref, brow_ref, bcol_ref, wl_ref, bl_ref,
                o_ref):
    w1 = wl_ref[0:_D, :]
    w2 = wl_ref[_D:2 * _D, :]
    w3 = wl_ref[2 * _D:3 * _D, :]
    w4 = wl_ref[3 * _D:4 * _D, :]
    h3 = h3_ref[...]
    t = (jnp.dot(h1_ref[...], w1, preferred_element_type=jnp.float32)
         + jnp.dot(h2_ref[...], w2, preferred_element_type=jnp.float32)
         + jnp.dot(h3, w3, preferred_element_type=jnp.float32))
    oh = (lax.broadcasted_iota(jnp.int32, (_G, _N), 0)
          == brow_ref[...]).astype(jnp.float32)
    pool = jnp.dot(oh, h3, preferred_element_type=jnp.float32)
    s = jnp.dot(pool, w4, preferred_element_type=jnp.float32)
    oht = (lax.broadcasted_iota(jnp.int32, (_N, _G), 1)
           == bcol_ref[...]).astype(jnp.float32)
    pooled = jnp.dot(oht, s, preferred_element_type=jnp.float32)
    z = t + pooled + bl_ref[...]
    o_ref[...] = 1.0 / (1.0 + jnp.exp(-z))


def _final(h1, h2, h3, batch, wl, bl):
    return pl.pallas_call(
        _final_body,
        out_shape=jax.ShapeDtypeStruct((_N, 1), jnp.float32),
    )(h1, h2, h3, batch.reshape(1, _N), batch.reshape(_N, 1), wl,
      bl.reshape(1, 1))


def kernel(x, edge_index, edge_attr, batch,
           W1e, b1e, W1, b1, g1, bt1,
           W2e, b2e, W2, b2, g2, bt2,
           W3e, b3e, W3, b3, g3, bt3,
           Wl, bl):
    src = edge_index[0]
    dst = edge_index[1]
    e1, e2, e3 = _edge_mlp(edge_attr, W1e, b1e, W2e, b2e, W3e, b3e)

    h = x
    hs = []
    for e_l, w, b, g, bt in ((e1, W1, b1, g1, bt1),
                             (e2, W2, b2, g2, bt2),
                             (e3, W3, b3, g3, bt3)):
        agg2 = _sc_edge(h, e_l, src, dst)
        h = _node(h, agg2, w, b, g, bt)
        hs.append(h)

    return _final(hs[0], hs[1], hs[2], batch, Wl, bl)
